# Initial kernel scaffold; baseline (speedup 1.0000x reference)
import jax
import jax.numpy as jnp
from jax.experimental import pallas as pl


def kernel(x, edge_index, batch, W_embed, b_embed, W_blocks, b_blocks, W_fc0, b_fc0, W_fc1, b_fc1):
    raise NotImplementedError("write your pallas kernel here")


# speedup vs baseline: 8.4224x; 8.4224x over previous
import functools

import jax
import jax.numpy as jnp
from jax import lax
from jax.experimental import pallas as pl
from jax.experimental.pallas import tpu as pltpu
from jax.experimental.pallas import tpu_sc as plsc

N = 10000
E = 320000
D_IN = 128
H = 128
FC_HID = 256
OUT = 64
G = 64

NC = 2
NS = 16
NW = NC * NS
CHUNK = 128
CH_PER_TILE = 80
E_PAD = NW * CH_PER_TILE * CHUNK
AGG_ROWS = 10240
ZR = 64
ROWS_OUT = 624

ROWB = 1000
NBLK = N // ROWB

@functools.cache
def _make_sc_kernel():
    mesh = plsc.VectorSubcoreMesh(core_axis_name="c", subcore_axis_name="s")
    return functools.partial(
        pl.kernel,
        mesh=mesh,
        out_type=jax.ShapeDtypeStruct((NC, N, H), jnp.float32),
        scratch_types=[
            pltpu.VMEM((CH_PER_TILE, CHUNK), jnp.int32),
            pltpu.VMEM((CH_PER_TILE, CHUNK), jnp.int32),
            pltpu.VMEM((CHUNK, H), jnp.float32),
            pltpu.VMEM((ZR, H), jnp.float32),
            pltpu.VMEM_SHARED((AGG_ROWS, H), jnp.float32),
            pltpu.SemaphoreType.DMA,
        ],
    )(_sc_gather_scatter)


def _sc_gather_scatter(h_hbm, src_hbm, dst_hbm, out_hbm,
                       src_v, dst_v, rows_v, zero_v, agg_sh, sem):
    cid = lax.axis_index("c")
    sid = lax.axis_index("s")
    wid = sid * NC + cid

    pltpu.sync_copy(src_hbm.at[pl.ds(wid * CH_PER_TILE, CH_PER_TILE)], src_v)
    pltpu.sync_copy(dst_hbm.at[pl.ds(wid * CH_PER_TILE, CH_PER_TILE)], dst_v)

    zv = jnp.zeros((16,), jnp.float32)

    def _zrow(i, carry):
        for c in range(H // 16):
            zero_v[i, pl.ds(c * 16, 16)] = zv
        return carry

    lax.fori_loop(0, ZR, _zrow, None)
    rows_per_tile = AGG_ROWS // NS

    def _zcopy(k, carry):
        pltpu.sync_copy(zero_v,
                        agg_sh.at[pl.ds(sid * rows_per_tile + k * ZR, ZR)])
        return carry

    lax.fori_loop(0, rows_per_tile // ZR, _zcopy, None)
    plsc.subcore_barrier()

    def _edge(j, carry):
        pltpu.async_copy(h_hbm.at[src_v.at[j]], rows_v, sem).wait()
        pltpu.sync_copy(rows_v, agg_sh.at[dst_v.at[j]], add=True)
        return carry

    lax.fori_loop(0, CH_PER_TILE, _edge, None)
    plsc.subcore_barrier()

    @pl.when(sid < NS - 1)
    def _():
        pltpu.sync_copy(agg_sh.at[pl.ds(sid * ROWS_OUT, ROWS_OUT)],
                        out_hbm.at[cid].at[pl.ds(sid * ROWS_OUT, ROWS_OUT)])

    @pl.when(sid == NS - 1)
    def _():
        last = (NS - 1) * ROWS_OUT
        pltpu.sync_copy(agg_sh.at[pl.ds(last, N - last)],
                        out_hbm.at[cid].at[pl.ds(last, N - last)])


def _embed_body(x_ref, w_ref, b_ref, o_ref):
    o_ref[...] = (jnp.dot(x_ref[...], w_ref[...],
                          preferred_element_type=jnp.float32) + b_ref[...])


def _embed(x, w, b):
    return pl.pallas_call(
        _embed_body,
        grid=(NBLK,),
        in_specs=[
            pl.BlockSpec((ROWB, D_IN), lambda i: (i, 0)),
            pl.BlockSpec((D_IN, H), lambda i: (0, 0)),
            pl.BlockSpec((1, H), lambda i: (0, 0)),
        ],
        out_specs=pl.BlockSpec((ROWB, H), lambda i: (i, 0)),
        out_shape=jax.ShapeDtypeStruct((N, H), jnp.float32),
    )(x, w, b)


def _block_body(p_ref, h_ref, w_ref, b_ref, o_ref):
    agg = p_ref[0] + p_ref[1]
    lin = jnp.dot(agg, w_ref[...], preferred_element_type=jnp.float32) + b_ref[...]
    o_ref[...] = h_ref[...] + jnp.maximum(lin, 0.0)


def _block_update(p, h, w, b):
    return pl.pallas_call(
        _block_body,
        grid=(NBLK,),
        in_specs=[
            pl.BlockSpec((NC, ROWB, H), lambda i: (0, i, 0)),
            pl.BlockSpec((ROWB, H), lambda i: (i, 0)),
            pl.BlockSpec((H, H), lambda i: (0, 0)),
            pl.BlockSpec((1, H), lambda i: (0, 0)),
        ],
        out_specs=pl.BlockSpec((ROWB, H), lambda i: (i, 0)),
        out_shape=jax.ShapeDtypeStruct((N, H), jnp.float32),
    )(p, h, w, b)


def _pool_head_body(h_ref, batch_ref, w0_ref, b0_ref, w1_ref, b1_ref,
                    o_ref, acc, cnt):
    i = pl.program_id(0)

    @pl.when(i == 0)
    def _():
        acc[...] = jnp.zeros_like(acc)
        cnt[...] = jnp.zeros_like(cnt)

    b = batch_ref[0]
    oh = (lax.broadcasted_iota(jnp.int32, (G, ROWB), 0) == b).astype(jnp.float32)
    acc[...] += jnp.dot(oh, h_ref[...], preferred_element_type=jnp.float32)
    cnt[...] += jnp.sum(oh, axis=1, keepdims=True)

    @pl.when(i == NBLK - 1)
    def _():
        pooled = acc[...] / jnp.maximum(cnt[...], 1.0)
        z = jnp.maximum(
            jnp.dot(pooled, w0_ref[...], preferred_element_type=jnp.float32)
            + b0_ref[...], 0.0)
        o_ref[...] = (jnp.dot(z, w1_ref[...], preferred_element_type=jnp.float32)
                      + b1_ref[...])


def _pool_head(h, batch3, w0, b0, w1, b1):
    return pl.pallas_call(
        _pool_head_body,
        grid=(NBLK,),
        in_specs=[
            pl.BlockSpec((ROWB, H), lambda i: (i, 0)),
            pl.BlockSpec((1, 1, ROWB), lambda i: (i, 0, 0)),
            pl.BlockSpec((H, FC_HID), lambda i: (0, 0)),
            pl.BlockSpec((1, FC_HID), lambda i: (0, 0)),
            pl.BlockSpec((FC_HID, OUT), lambda i: (0, 0)),
            pl.BlockSpec((1, OUT), lambda i: (0, 0)),
        ],
        out_specs=pl.BlockSpec((G, OUT), lambda i: (0, 0)),
        out_shape=jax.ShapeDtypeStruct((G, OUT), jnp.float32),
        scratch_shapes=[
            pltpu.VMEM((G, H), jnp.float32),
            pltpu.VMEM((G, 1), jnp.float32),
        ],
    )(h, batch3, w0, b0, w1, b1)


def kernel(x, edge_index, batch, W_embed, b_embed, W_blocks, b_blocks,
           W_fc0, b_fc0, W_fc1, b_fc1):
    pad = E_PAD - E
    src = jnp.concatenate([edge_index[0],
                           lax.iota(jnp.int32, pad) % 512])
    dst = jnp.concatenate([edge_index[1],
                           N + (lax.iota(jnp.int32, pad) % 8)])
    src2 = src.reshape(NW * CH_PER_TILE, CHUNK)
    dst2 = dst.reshape(NW * CH_PER_TILE, CHUNK)
    batch3 = batch.reshape(NBLK, 1, ROWB)

    sc_agg = _make_sc_kernel()
    h = _embed(x, W_embed, b_embed.reshape(1, H))
    for i in range(3):
        p = sc_agg(h, src2, dst2)
        h = _block_update(p, h, W_blocks[i], b_blocks[i].reshape(1, H))
    return _pool_head(h, batch3, W_fc0, b_fc0.reshape(1, FC_HID),
                      W_fc1, b_fc1.reshape(1, OUT))


# speedup vs baseline: 11.8360x; 1.4053x over previous
import functools

import jax
import jax.numpy as jnp
from jax import lax
from jax.experimental import pallas as pl
from jax.experimental.pallas import tpu as pltpu
from jax.experimental.pallas import tpu_sc as plsc

N = 10000
E = 320000
D_IN = 128
H = 128
FC_HID = 256
OUT = 64
G = 64

NC = 2
NS = 16
NW = NC * NS
CHUNK = 128
CH_PER_TILE = 80
E_PAD = NW * CH_PER_TILE * CHUNK
AGG_ROWS = 10240
ZR = 64
ROWS_OUT = 624

ROWB = 1000
NBLK = N // ROWB

SUP = 8
NSUP = CH_PER_TILE // SUP


@functools.cache
def _make_sc_kernel():
    mesh = plsc.VectorSubcoreMesh(core_axis_name="c", subcore_axis_name="s")
    return functools.partial(
        pl.kernel,
        mesh=mesh,
        out_type=jax.ShapeDtypeStruct((NC, N, H), jnp.float32),
        scratch_types=[
            pltpu.VMEM((2, SUP, CHUNK), jnp.int32),
            pltpu.VMEM((2, SUP, CHUNK), jnp.int32),
            pltpu.VMEM((2, CHUNK, H), jnp.float32),
            pltpu.VMEM((ZR, H), jnp.float32),
            pltpu.VMEM_SHARED((AGG_ROWS, H), jnp.float32),
        ] + [pltpu.SemaphoreType.DMA] * 6,
    )(_sc_gather_scatter)


def _sc_gather_scatter(h_hbm, src_hbm, dst_hbm, out_hbm,
                       src_v, dst_v, rows_v, zero_v, agg_sh,
                       g0, g1, s0, s1, i0, i1):
    gsems = (g0, g1)
    ssems = (s0, s1)
    isems = (i0, i1)
    cid = lax.axis_index("c")
    sid = lax.axis_index("s")
    wid = sid * NC + cid
    row0 = wid * CH_PER_TILE

    def idx_start(p, t):
        pltpu.async_copy(src_hbm.at[pl.ds(row0 + t * SUP, SUP)],
                         src_v.at[p], isems[p])
        pltpu.async_copy(dst_hbm.at[pl.ds(row0 + t * SUP, SUP)],
                         dst_v.at[p], isems[p])

    def idx_wait(p):
        pltpu.make_async_copy(src_hbm.at[pl.ds(0, SUP)], src_v.at[p],
                              isems[p]).wait()
        pltpu.make_async_copy(dst_hbm.at[pl.ds(0, SUP)], dst_v.at[p],
                              isems[p]).wait()

    def gather_start(slot, p, b):
        pltpu.async_copy(h_hbm.at[src_v.at[p].at[b]], rows_v.at[slot],
                         gsems[slot])

    def gather_wait(slot):
        pltpu.make_async_copy(h_hbm.at[src_v.at[0].at[0]], rows_v.at[slot],
                              gsems[slot]).wait()

    def scatter_start(slot, p, b):
        pltpu.async_copy(rows_v.at[slot], agg_sh.at[dst_v.at[p].at[b]],
                         ssems[slot], add=True)

    def scatter_wait(slot):
        pltpu.make_async_copy(rows_v.at[slot], agg_sh.at[dst_v.at[0].at[0]],
                              ssems[slot]).wait()

    idx_start(0, 0)
    idx_start(1, 1)

    zv = jnp.zeros((16,), jnp.float32)

    def _zrow(i, carry):
        for c in range(H // 16):
            zero_v[i, pl.ds(c * 16, 16)] = zv
        return carry

    lax.fori_loop(0, ZR, _zrow, None)
    rows_per_tile = AGG_ROWS // NS

    def _zcopy(k, carry):
        pltpu.sync_copy(zero_v,
                        agg_sh.at[pl.ds(sid * rows_per_tile + k * ZR, ZR)])
        return carry

    lax.fori_loop(0, rows_per_tile // ZR, _zcopy, None)
    plsc.subcore_barrier()

    def do_super(t, p, prefetch):
        idx_wait(p)
        gather_start(0, p, 0)
        gather_start(1, p, 1)
        for b in range(SUP):
            slot = b & 1
            gather_wait(slot)
            scatter_start(slot, p, b)
            if b < SUP - 2:
                scatter_wait(slot)
                gather_start(slot, p, b + 2)
        scatter_wait(0)
        scatter_wait(1)
        if prefetch:
            idx_start(p, t + 2)

    def pair_body(k, carry):
        do_super(2 * k, 0, True)
        do_super(2 * k + 1, 1, True)
        return carry

    lax.fori_loop(0, NSUP // 2 - 1, pair_body, None)
    do_super(NSUP - 2, 0, False)
    do_super(NSUP - 1, 1, False)
    plsc.subcore_barrier()

    @pl.when(sid < NS - 1)
    def _():
        pltpu.sync_copy(agg_sh.at[pl.ds(sid * ROWS_OUT, ROWS_OUT)],
                        out_hbm.at[cid].at[pl.ds(sid * ROWS_OUT, ROWS_OUT)])

    @pl.when(sid == NS - 1)
    def _():
        last = (NS - 1) * ROWS_OUT
        pltpu.sync_copy(agg_sh.at[pl.ds(last, N - last)],
                        out_hbm.at[cid].at[pl.ds(last, N - last)])


def _embed_body(x_ref, w_ref, b_ref, o_ref):
    o_ref[...] = (jnp.dot(x_ref[...], w_ref[...],
                          preferred_element_type=jnp.float32) + b_ref[...])


def _embed(x, w, b):
    return pl.pallas_call(
        _embed_body,
        grid=(NBLK,),
        in_specs=[
            pl.BlockSpec((ROWB, D_IN), lambda i: (i, 0)),
            pl.BlockSpec((D_IN, H), lambda i: (0, 0)),
            pl.BlockSpec((1, H), lambda i: (0, 0)),
        ],
        out_specs=pl.BlockSpec((ROWB, H), lambda i: (i, 0)),
        out_shape=jax.ShapeDtypeStruct((N, H), jnp.float32),
    )(x, w, b)


def _block_body(p_ref, h_ref, w_ref, b_ref, o_ref):
    agg = p_ref[0] + p_ref[1]
    lin = jnp.dot(agg, w_ref[...], preferred_element_type=jnp.float32) + b_ref[...]
    o_ref[...] = h_ref[...] + jnp.maximum(lin, 0.0)


def _block_update(p, h, w, b):
    return pl.pallas_call(
        _block_body,
        grid=(NBLK,),
        in_specs=[
            pl.BlockSpec((NC, ROWB, H), lambda i: (0, i, 0)),
            pl.BlockSpec((ROWB, H), lambda i: (i, 0)),
            pl.BlockSpec((H, H), lambda i: (0, 0)),
            pl.BlockSpec((1, H), lambda i: (0, 0)),
        ],
        out_specs=pl.BlockSpec((ROWB, H), lambda i: (i, 0)),
        out_shape=jax.ShapeDtypeStruct((N, H), jnp.float32),
    )(p, h, w, b)


def _pool_head_body(h_ref, batch_ref, w0_ref, b0_ref, w1_ref, b1_ref,
                    o_ref, acc, cnt):
    i = pl.program_id(0)

    @pl.when(i == 0)
    def _():
        acc[...] = jnp.zeros_like(acc)
        cnt[...] = jnp.zeros_like(cnt)

    b = batch_ref[0]
    oh = (lax.broadcasted_iota(jnp.int32, (G, ROWB), 0) == b).astype(jnp.float32)
    acc[...] += jnp.dot(oh, h_ref[...], preferred_element_type=jnp.float32)
    cnt[...] += jnp.sum(oh, axis=1, keepdims=True)

    @pl.when(i == NBLK - 1)
    def _():
        pooled = acc[...] / jnp.maximum(cnt[...], 1.0)
        z = jnp.maximum(
            jnp.dot(pooled, w0_ref[...], preferred_element_type=jnp.float32)
            + b0_ref[...], 0.0)
        o_ref[...] = (jnp.dot(z, w1_ref[...], preferred_element_type=jnp.float32)
                      + b1_ref[...])


def _pool_head(h, batch3, w0, b0, w1, b1):
    return pl.pallas_call(
        _pool_head_body,
        grid=(NBLK,),
        in_specs=[
            pl.BlockSpec((ROWB, H), lambda i: (i, 0)),
            pl.BlockSpec((1, 1, ROWB), lambda i: (i, 0, 0)),
            pl.BlockSpec((H, FC_HID), lambda i: (0, 0)),
            pl.BlockSpec((1, FC_HID), lambda i: (0, 0)),
            pl.BlockSpec((FC_HID, OUT), lambda i: (0, 0)),
            pl.BlockSpec((1, OUT), lambda i: (0, 0)),
        ],
        out_specs=pl.BlockSpec((G, OUT), lambda i: (0, 0)),
        out_shape=jax.ShapeDtypeStruct((G, OUT), jnp.float32),
        scratch_shapes=[
            pltpu.VMEM((G, H), jnp.float32),
            pltpu.VMEM((G, 1), jnp.float32),
        ],
    )(h, batch3, w0, b0, w1, b1)


def kernel(x, edge_index, batch, W_embed, b_embed, W_blocks, b_blocks,
           W_fc0, b_fc0, W_fc1, b_fc1):
    pad = E_PAD - E
    src = jnp.concatenate([edge_index[0],
                           lax.iota(jnp.int32, pad) % 512])
    dst = jnp.concatenate([edge_index[1],
                           N + (lax.iota(jnp.int32, pad) % 8)])
    src2 = src.reshape(NW * CH_PER_TILE, CHUNK)
    dst2 = dst.reshape(NW * CH_PER_TILE, CHUNK)
    batch3 = batch.reshape(NBLK, 1, ROWB)

    sc_agg = _make_sc_kernel()
    h = _embed(x, W_embed, b_embed.reshape(1, H))
    for i in range(3):
        p = sc_agg(h, src2, dst2)
        h = _block_update(p, h, W_blocks[i], b_blocks[i].reshape(1, H))
    return _pool_head(h, batch3, W_fc0, b_fc0.reshape(1, FC_HID),
                      W_fc1, b_fc1.reshape(1, OUT))
